# baseline (speedup 1.0000x reference)
import jax
import jax.numpy as jnp
from jax.experimental import pallas as pl
from jax.experimental.pallas import tpu as pltpu

_BULK_CHUNKS_PER_BATCH = 2


def _body(xs_ref, x_ref, xv_ref, xvt_ref, yv_ref, y_ref, out_ref,
          rows_ref, bulk_sem, gat_sem, sct_sem):
    B, H, W = y_ref.shape
    n = xs_ref.shape[0]
    hc = H // _BULK_CHUNKS_PER_BATCH

    bulk = []
    for b in range(B):
        for c in range(_BULK_CHUNKS_PER_BATCH):
            cp = pltpu.make_async_copy(
                y_ref.at[b, pl.ds(c * hc, hc), :],
                out_ref.at[b, pl.ds(c * hc, hc), :],
                bulk_sem,
            )
            cp.start()
            bulk.append(cp)

    gathers = []
    for i in range(n):
        xi = xs_ref[i]
        cp = pltpu.make_async_copy(
            y_ref.at[:, pl.ds(xi, 1), :],
            rows_ref.at[:, pl.ds(i, 1), :],
            gat_sem,
        )
        cp.start()
        gathers.append(cp)
    for cp in gathers:
        cp.wait()

    match = (xv_ref[...] == xvt_ref[...]).astype(jnp.float32)
    col_iota = jax.lax.broadcasted_iota(jnp.int32, (n, W), 1)
    onehot = (col_iota == yv_ref[...]).astype(jnp.float32)
    count = jax.lax.dot(match, onehot, preferred_element_type=jnp.float32)
    rows_ref[...] = rows_ref[...] + x_ref[...][:, :, None] * count[None]

    for cp in bulk:
        cp.wait()
    scatters = []
    for i in range(n):
        xi = xs_ref[i]
        cp = pltpu.make_async_copy(
            rows_ref.at[:, pl.ds(i, 1), :],
            out_ref.at[:, pl.ds(xi, 1), :],
            sct_sem,
        )
        cp.start()
        scatters.append(cp)
    for cp in scatters:
        cp.wait()


@jax.jit
def kernel(Y, X, x_idx, y_idx):
    B, H, W = Y.shape
    n = x_idx.shape[0]
    return pl.pallas_call(
        _body,
        in_specs=[
            pl.BlockSpec(memory_space=pltpu.MemorySpace.SMEM),
            pl.BlockSpec(memory_space=pltpu.MemorySpace.VMEM),
            pl.BlockSpec(memory_space=pltpu.MemorySpace.VMEM),
            pl.BlockSpec(memory_space=pltpu.MemorySpace.VMEM),
            pl.BlockSpec(memory_space=pltpu.MemorySpace.VMEM),
            pl.BlockSpec(memory_space=pltpu.MemorySpace.HBM),
        ],
        out_specs=pl.BlockSpec(memory_space=pltpu.MemorySpace.HBM),
        out_shape=jax.ShapeDtypeStruct((B, H, W), Y.dtype),
        scratch_shapes=[
            pltpu.VMEM((B, n, W), jnp.float32),
            pltpu.SemaphoreType.DMA,
            pltpu.SemaphoreType.DMA,
            pltpu.SemaphoreType.DMA,
        ],
    )(x_idx, X, x_idx.reshape(n, 1), x_idx.reshape(1, n),
      y_idx.reshape(n, 1), Y)


# speedup vs baseline: 47.8220x; 47.8220x over previous
import jax
import jax.numpy as jnp
from jax.experimental import pallas as pl
from jax.experimental.pallas import tpu as pltpu

_BR = 1024


def _body(x_ref, xi_ref, yi_ref, y_ref, out_ref):
    b = pl.program_id(0)
    r = pl.program_id(1)
    row_start = r * _BR

    blk = y_ref[0]
    n = xi_ref.shape[1]
    cols = blk.shape[1]

    row_iota = jax.lax.broadcasted_iota(jnp.int32, (_BR, n), 0) + row_start
    rowsel = (row_iota == xi_ref[0][None, :]).astype(jnp.float32)
    col_iota = jax.lax.broadcasted_iota(jnp.int32, (n, cols), 1)
    onehot = (col_iota == yi_ref[0][:, None]).astype(jnp.float32)

    add = jax.lax.dot(rowsel, onehot, preferred_element_type=jnp.float32)
    out_ref[0] = blk + x_ref[b, 0] * add


@jax.jit
def kernel(Y, X, x_idx, y_idx):
    B, H, W = Y.shape
    n = x_idx.shape[0]
    grid = (B, H // _BR)
    return pl.pallas_call(
        _body,
        grid=grid,
        in_specs=[
            pl.BlockSpec(memory_space=pltpu.SMEM),
            pl.BlockSpec((1, n), lambda b, r: (0, 0)),
            pl.BlockSpec((1, n), lambda b, r: (0, 0)),
            pl.BlockSpec((1, _BR, W), lambda b, r: (b, r, 0)),
        ],
        out_specs=pl.BlockSpec((1, _BR, W), lambda b, r: (b, r, 0)),
        out_shape=jax.ShapeDtypeStruct((B, H, W), Y.dtype),
        compiler_params=pltpu.CompilerParams(
            dimension_semantics=("parallel", "parallel"),
        ),
    )(X, x_idx.reshape(1, n), y_idx.reshape(1, n), Y)
